# Initial kernel scaffold; baseline (speedup 1.0000x reference)
import jax
import jax.numpy as jnp
from jax.experimental import pallas as pl


def kernel(x, edge_index, g1, b1, W1, asrc1, adst1, bb1, g2, b2, W2, asrc2, adst2, bb2, g3, b3, Ws, bs, Wp1, bp1, Wp2, bp2):
    raise NotImplementedError("write your pallas kernel here")


# speedup vs baseline: 56.0218x; 56.0218x over previous
import functools

import jax
import jax.numpy as jnp
import numpy as np
from jax import lax
from jax.experimental import pallas as pl
from jax.experimental.pallas import tpu as pltpu
from jax.experimental.pallas import tpu_sc as plsc

N = 10000
D = 128
H = 8
C = 16
HC = H * C
OUT = 1

NW = 32
NSUB = 16
CHUNK = 128
CPT = 81
E_TOT = 320000 + N
E_PAD = NW * CPT * CHUNK
N_PAD = 10112
RPT = N_PAD // NSUB
TRASH = N

BLK = 2000
NBLK = N // BLK

_f32 = jnp.float32


def _stats_body(x_ref, s_ref, q_ref):
    i = pl.program_id(0)

    @pl.when(i == 0)
    def _():
        s_ref[...] = jnp.zeros_like(s_ref)
        q_ref[...] = jnp.zeros_like(q_ref)

    x = x_ref[...]
    s_ref[...] += jnp.sum(x, axis=0, keepdims=True)
    q_ref[...] += jnp.sum(x * x, axis=0, keepdims=True)


def _bn_mm_scores_body(x_ref, s_ref, q_ref, w_ref, g_ref, b_ref,
                       af_ref, ad_ref, eh_ref, h_ref, ss_ref, sd_ref):
    mean = s_ref[...] / N
    var = q_ref[...] / N - mean * mean
    xn = (x_ref[...] - mean) * lax.rsqrt(var + 1e-5) * g_ref[...] + b_ref[...]
    ht = jnp.dot(xn, w_ref[...], preferred_element_type=_f32)
    h_ref[...] = ht
    ss_ref[...] = jnp.dot(ht * af_ref[...], eh_ref[...],
                          preferred_element_type=_f32)
    sd_ref[...] = jnp.dot(ht * ad_ref[...], eh_ref[...],
                          preferred_element_type=_f32)


def _combine_act_stats_body(a0_ref, a1_ref, d0_ref, d1_ref, bb_ref, sp_ref,
                            act_ref, s_ref, q_ref):
    i = pl.program_id(0)

    @pl.when(i == 0)
    def _():
        s_ref[...] = jnp.zeros_like(s_ref)
        q_ref[...] = jnp.zeros_like(q_ref)

    num = a0_ref[...] + a1_ref[...]
    den = jnp.dot(d0_ref[...] + d1_ref[...], sp_ref[...],
                  preferred_element_type=_f32) + 1e-16
    g = num / den + bb_ref[...]
    act = jnp.where(g > 0, g, jnp.exp(jnp.minimum(g, 0.0)) - 1.0)
    act_ref[...] = act
    s_ref[...] += jnp.sum(act, axis=0, keepdims=True)
    q_ref[...] += jnp.sum(act * act, axis=0, keepdims=True)


def _head_body(act_ref, s_ref, q_ref, g_ref, b_ref, x_ref, ws_ref, bs_ref,
               wp1_ref, bp1_ref, wp2_ref, bp2_ref, o_ref):
    mean = s_ref[...] / N
    var = q_ref[...] / N - mean * mean
    h = (act_ref[...] - mean) * lax.rsqrt(var + 1e-5) * g_ref[...] + b_ref[...]
    h = h + jnp.dot(x_ref[...], ws_ref[...], preferred_element_type=_f32)
    h = h + bs_ref[...]
    r = jnp.dot(h, wp1_ref[...], preferred_element_type=_f32) + bp1_ref[...]
    r = jnp.maximum(r, 0.0)
    o_ref[...] = jnp.dot(r, wp2_ref[...], preferred_element_type=_f32) + bp2_ref[...]


def _row_spec(w):
    return pl.BlockSpec((BLK, w), lambda i: (i, 0))


def _full_spec(r, w):
    return pl.BlockSpec((r, w), lambda i: (0, 0))


def _stats(x):
    return pl.pallas_call(
        _stats_body,
        grid=(NBLK,),
        in_specs=[_row_spec(D)],
        out_specs=[_full_spec(1, D), _full_spec(1, D)],
        out_shape=[jax.ShapeDtypeStruct((1, D), _f32)] * 2,
    )(x)


def _bn_mm_scores(x, s, q, w, g, b, af, ad, eh):
    return pl.pallas_call(
        _bn_mm_scores_body,
        grid=(NBLK,),
        in_specs=[_row_spec(D), _full_spec(1, D), _full_spec(1, D),
                  _full_spec(D, HC), _full_spec(1, D), _full_spec(1, D),
                  _full_spec(1, HC), _full_spec(1, HC), _full_spec(HC, 16)],
        out_specs=[_row_spec(HC), _row_spec(16), _row_spec(16)],
        out_shape=[jax.ShapeDtypeStruct((N, HC), _f32),
                   jax.ShapeDtypeStruct((N, 16), _f32),
                   jax.ShapeDtypeStruct((N, 16), _f32)],
    )(x, s, q, w, g, b, af, ad, eh)


def _combine_act_stats(acc, den, bb, sp):
    return pl.pallas_call(
        _combine_act_stats_body,
        grid=(NBLK,),
        in_specs=[_row_spec(HC), _row_spec(HC), _row_spec(16), _row_spec(16),
                  _full_spec(1, HC), _full_spec(16, HC)],
        out_specs=[_row_spec(HC), _full_spec(1, HC), _full_spec(1, HC)],
        out_shape=[jax.ShapeDtypeStruct((N, HC), _f32),
                   jax.ShapeDtypeStruct((1, HC), _f32),
                   jax.ShapeDtypeStruct((1, HC), _f32)],
    )(acc[0, :N], acc[1, :N], den[0, :N], den[1, :N], bb, sp)


def _head(act, s, q, g, b, x, ws, bs, wp1, bp1, wp2, bp2):
    return pl.pallas_call(
        _head_body,
        grid=(NBLK,),
        in_specs=[_row_spec(HC), _full_spec(1, HC), _full_spec(1, HC),
                  _full_spec(1, HC), _full_spec(1, HC), _row_spec(D),
                  _full_spec(D, HC), _full_spec(1, HC),
                  _full_spec(HC, C), _full_spec(1, C),
                  _full_spec(C, OUT), _full_spec(1, OUT)],
        out_specs=[_row_spec(OUT)],
        out_shape=[jax.ShapeDtypeStruct((N, OUT), _f32)],
    )(act, s, q, g, b, x, ws, bs, wp1, bp1, wp2, bp2)


def _sc_edge_body(htab, ssrc, sdst, sidx, didx, z128, z16,
                  acc_out, den_out,
                  acc_sh, den_sh, sidx_c, didx_c,
                  rows_v, gs_v, gd_v, e_v):
    c = lax.axis_index("c")
    s = lax.axis_index("s")
    wid = c * NSUB + s

    pltpu.sync_copy(z128.at[pl.ds(s * RPT, RPT)], acc_sh.at[pl.ds(s * RPT, RPT)])
    pltpu.sync_copy(z16.at[pl.ds(s * RPT, RPT)], den_sh.at[pl.ds(s * RPT, RPT)])

    plsc.subcore_barrier()

    lane = lax.iota(jnp.int32, 16)
    head_mask = lane < 8

    def chunk_body(g, carry):
        pltpu.sync_copy(sidx.at[wid, g], sidx_c)
        pltpu.sync_copy(didx.at[wid, g], didx_c)
        pltpu.sync_copy(htab.at[sidx_c], rows_v)
        pltpu.sync_copy(ssrc.at[sidx_c], gs_v)
        pltpu.sync_copy(sdst.at[didx_c], gd_v)

        def edge_body(k, carry2):
            a = gs_v[k, :] + gd_v[k, :]
            a = jnp.where(a >= 0, a, a * jnp.float32(0.2))
            e = jnp.where(head_mask, jnp.exp(a), jnp.float32(0.0))
            e_v[k, :] = e
            for j in range(H):
                rows_v[k, pl.ds(j * C, C)] = rows_v[k, pl.ds(j * C, C)] * e[j]
            return carry2

        lax.fori_loop(0, CHUNK, edge_body, 0, unroll=False)

        pltpu.sync_copy(rows_v, acc_sh.at[didx_c], add=True)
        pltpu.sync_copy(e_v, den_sh.at[didx_c], add=True)
        return carry

    lax.fori_loop(0, CPT, chunk_body, 0, unroll=False)

    plsc.subcore_barrier()

    pltpu.sync_copy(acc_sh.at[pl.ds(s * RPT, RPT)],
                    acc_out.at[c, pl.ds(s * RPT, RPT)])
    pltpu.sync_copy(den_sh.at[pl.ds(s * RPT, RPT)],
                    den_out.at[c, pl.ds(s * RPT, RPT)])


def _sc_edge(htab, ssrc, sdst, sidx, didx, z128, z16):
    fn = pl.kernel(
        _sc_edge_body,
        out_type=(jax.ShapeDtypeStruct((2, N_PAD, HC), _f32),
                  jax.ShapeDtypeStruct((2, N_PAD, 16), _f32)),
        mesh=plsc.VectorSubcoreMesh(core_axis_name="c", subcore_axis_name="s"),
        compiler_params=pltpu.CompilerParams(use_tc_tiling_on_sc=False),
        scratch_types=(
            pltpu.VMEM_SHARED((N_PAD, HC), _f32),
            pltpu.VMEM_SHARED((N_PAD, 16), _f32),
            pltpu.VMEM((CHUNK,), jnp.int32),
            pltpu.VMEM((CHUNK,), jnp.int32),
            pltpu.VMEM((CHUNK, HC), _f32),
            pltpu.VMEM((CHUNK, 16), _f32),
            pltpu.VMEM((CHUNK, 16), _f32),
            pltpu.VMEM((CHUNK, 16), _f32),
        ),
    )
    return fn(htab, ssrc, sdst, sidx, didx, z128, z16)


def _head_pool():
    m = np.zeros((HC, 16), np.float32)
    for h in range(H):
        m[h * C:(h + 1) * C, h] = 1.0
    return jnp.asarray(m)


def _head_spread():
    m = np.zeros((16, HC), np.float32)
    for h in range(H):
        m[h, h * C:(h + 1) * C] = 1.0
    return jnp.asarray(m)


def kernel(x, edge_index, g1, b1, W1, asrc1, adst1, bb1, g2, b2, W2, asrc2,
           adst2, bb2, g3, b3, Ws, bs, Wp1, bp1, Wp2, bp2):
    eh = _head_pool()
    sp = _head_spread()

    loop = jnp.arange(N, dtype=edge_index.dtype)
    src = jnp.concatenate([edge_index[0], loop,
                           jnp.zeros((E_PAD - E_TOT,), edge_index.dtype)])
    dst = jnp.concatenate([edge_index[1], loop,
                           jnp.full((E_PAD - E_TOT,), TRASH, edge_index.dtype)])
    sidx = src.reshape(NW, CPT, CHUNK)
    didx = dst.reshape(NW, CPT, CHUNK)

    z128 = jnp.zeros((N_PAD, HC), _f32)
    z16 = jnp.zeros((N_PAD, 16), _f32)
    spad = jnp.full((N_PAD - N, 16), -1e30, _f32)

    def gat_layer(feat, s, q, g, b, W, af, ad):
        h1, ss, sd = _bn_mm_scores(feat, s, q, W, g, b, af, ad, eh)
        sd_full = jnp.concatenate([sd, spad], axis=0)
        return _sc_edge(h1, ss, sd_full, sidx, didx, z128, z16)

    s1, q1 = _stats(x)
    acc1, den1 = gat_layer(x, s1, q1, g1.reshape(1, D), b1.reshape(1, D), W1,
                           asrc1.reshape(1, HC), adst1.reshape(1, HC))
    act1, s2, q2 = _combine_act_stats(acc1, den1, bb1.reshape(1, HC), sp)

    acc2, den2 = gat_layer(act1, s2, q2, g2.reshape(1, HC), b2.reshape(1, HC),
                           W2, asrc2.reshape(1, HC), adst2.reshape(1, HC))
    act2, s3, q3 = _combine_act_stats(acc2, den2, bb2.reshape(1, HC), sp)

    out = _head(act2, s3, q3, g3.reshape(1, HC), b3.reshape(1, HC), x, Ws,
                bs.reshape(1, HC), Wp1, bp1.reshape(1, C), Wp2,
                bp2.reshape(1, OUT))
    return out[0]
